# baseline (speedup 1.0000x reference)
import functools

import jax
from jax import lax
from jax.experimental import pallas as pl
from jax.experimental.pallas import tpu as pltpu
from jax.experimental.pallas import tpu_sc as plsc

_NC = 2
_NS = 16
_CH = 16
_NBUF = 2


def kernel(x, table):
    batch, length = x.shape
    feat = table.shape[1]
    n_workers = _NC * _NS
    rows_per_w = length // n_workers
    n_ch = rows_per_w // _CH
    mesh = plsc.VectorSubcoreMesh(core_axis_name="c", subcore_axis_name="s")

    @functools.partial(
        pl.kernel,
        mesh=mesh,
        out_type=jax.ShapeDtypeStruct((batch, length, feat), table.dtype),
        scratch_types=[
            pltpu.VMEM((_NBUF, _CH, feat), table.dtype),
            pltpu.SemaphoreType.DMA((_NBUF,)),
            pltpu.SemaphoreType.DMA((_NBUF,)),
        ],
    )
    def sc_broadcast(table_hbm, out_hbm, buf, sem_in, sem_out):
        wid = lax.axis_index("c") * _NS + lax.axis_index("s")
        base = wid * rows_per_w

        def in_copy(ch, slot):
            return pltpu.make_async_copy(
                table_hbm.at[pl.ds(base + ch * _CH, _CH)], buf.at[slot], sem_in.at[slot]
            )

        def out_copy(ch, slot, b):
            return pltpu.make_async_copy(
                buf.at[slot], out_hbm.at[b, pl.ds(base + ch * _CH, _CH)], sem_out.at[slot]
            )

        for ch in range(min(_NBUF, n_ch)):
            in_copy(ch, ch).start()
        for ch in range(n_ch):
            slot = ch % _NBUF
            in_copy(ch, slot).wait()
            for b in range(batch):
                out_copy(ch, slot, b).start()
            nxt = ch + 1
            if _NBUF <= nxt < n_ch:
                for b in range(batch):
                    out_copy(nxt - _NBUF, nxt % _NBUF, b).wait()
                in_copy(nxt, nxt % _NBUF).start()
        for ch in range(max(0, n_ch - _NBUF), n_ch):
            for b in range(batch):
                out_copy(ch, ch % _NBUF, b).wait()

    return sc_broadcast(table)


# speedup vs baseline: 1.0032x; 1.0032x over previous
import functools

import jax
from jax import lax
from jax.experimental import pallas as pl
from jax.experimental.pallas import tpu as pltpu
from jax.experimental.pallas import tpu_sc as plsc

_NC = 2
_NS = 16
_CH = 16
_NBUF = 2


def kernel(x, table):
    batch, length = x.shape
    feat = table.shape[1]
    n_workers = _NC * _NS
    rows_per_w = length // n_workers
    n_ch = rows_per_w // _CH
    mesh = plsc.VectorSubcoreMesh(core_axis_name="c", subcore_axis_name="s")

    @functools.partial(
        pl.kernel,
        mesh=mesh,
        out_type=jax.ShapeDtypeStruct((batch, length, feat), table.dtype),
        scratch_types=[
            pltpu.VMEM((_NBUF, _CH, feat), table.dtype),
            pltpu.SemaphoreType.DMA((_NBUF,)),
            pltpu.SemaphoreType.DMA((_NBUF,)),
        ],
    )
    def sc_broadcast(table_hbm, out_hbm, buf, sem_in, sem_out):
        wid = lax.axis_index("s") * _NC + lax.axis_index("c")
        base = wid * rows_per_w

        def in_copy(ch, slot):
            return pltpu.make_async_copy(
                table_hbm.at[pl.ds(base + ch * _CH, _CH)], buf.at[slot], sem_in.at[slot]
            )

        def out_copy(ch, slot, b):
            return pltpu.make_async_copy(
                buf.at[slot], out_hbm.at[b, pl.ds(base + ch * _CH, _CH)], sem_out.at[slot]
            )

        for ch in range(min(_NBUF, n_ch)):
            in_copy(ch, ch).start()
        for ch in range(n_ch):
            slot = ch % _NBUF
            in_copy(ch, slot).wait()
            for k in range(batch):
                out_copy(ch, slot, (k + ch) % batch).start()
            nxt = ch + 1
            if _NBUF <= nxt < n_ch:
                for b in range(batch):
                    out_copy(nxt - _NBUF, nxt % _NBUF, b).wait()
                in_copy(nxt, nxt % _NBUF).start()
        for ch in range(max(0, n_ch - _NBUF), n_ch):
            for b in range(batch):
                out_copy(ch, ch % _NBUF, b).wait()

    return sc_broadcast(table)
